# baseline (speedup 1.0000x reference)
import jax
import jax.numpy as jnp
from jax.experimental import pallas as pl
from jax.experimental.pallas import tpu as pltpu
from jax.experimental.pallas import tpu_sc as plsc

_VOCAB = 1000000
_D = 64
_NS = 2048
_BR = 1024

_VPAD = 1024 * 1024


def _gumbel_padded():
    u = jax.random.uniform(
        jax.random.key(42), (_VOCAB,), minval=1e-10, maxval=1.0
    )
    g = -jnp.log(-jnp.log(u))
    return jnp.full((_VPAD,), -1e30, jnp.float32).at[:_VOCAB].set(g)


_ROWS = 1024
_COLS = 1024
_NCHUNK = 32


def _select_body(probs_ref, gumbel_ref, tpos_ref):
    z = jnp.log(probs_ref[...] + 1e-10) + gumbel_ref[...]
    b = jax.lax.bitcast_convert_type(z, jnp.uint32)
    key = jnp.where((b >> 31) == 1, ~b, b | jnp.uint32(0x80000000))

    row = jax.lax.broadcasted_iota(jnp.int32, (_ROWS, _COLS), 0)
    col = jax.lax.broadcasted_iota(jnp.int32, (_ROWS, _COLS), 1)
    idx = row * _COLS + col

    def bit_body(i, t):
        cand = t | jax.lax.shift_left(
            jnp.uint32(1), (31 - i).astype(jnp.uint32)
        )
        cnt = jnp.sum((key >= cand).astype(jnp.int32))
        return jnp.where(cnt >= _NS, cand, t)

    tval = jax.lax.fori_loop(0, 32, bit_body, jnp.uint32(0))

    need = _NS - jnp.sum((key > tval).astype(jnp.int32))

    def tie_body(j, iv):
        cand = iv | jax.lax.shift_left(jnp.int32(1), 19 - j)
        cnt = jnp.sum(((key == tval) & (idx <= cand)).astype(jnp.int32))
        return jnp.where(cnt <= need, cand, iv)

    ival = jax.lax.fori_loop(0, 20, tie_body, jnp.int32(0))

    sel = (key > tval) | ((key == tval) & (idx <= ival))
    self_f = sel.astype(jnp.float32)
    rowsum = jnp.sum(self_f, axis=1, keepdims=True)
    utri = (
        jax.lax.broadcasted_iota(jnp.int32, (_COLS, _COLS), 0)
        <= jax.lax.broadcasted_iota(jnp.int32, (_COLS, _COLS), 1)
    ).astype(jnp.float32)
    rowcum = jnp.dot(self_f, utri, preferred_element_type=jnp.float32)
    ltri = (
        jax.lax.broadcasted_iota(jnp.int32, (_ROWS, _ROWS), 0)
        > jax.lax.broadcasted_iota(jnp.int32, (_ROWS, _ROWS), 1)
    ).astype(jnp.float32)
    rowoff = jnp.dot(ltri, rowsum, preferred_element_type=jnp.float32)
    grank = (rowoff + rowcum - 1.0).astype(jnp.int32)
    tpos_ref[...] = jnp.where(sel, grank, _NS)


def _select(probs2d, gumbel2d):
    return pl.pallas_call(
        _select_body,
        grid=(1,),
        in_specs=[
            pl.BlockSpec((_ROWS, _COLS), lambda i: (0, 0)),
            pl.BlockSpec((_ROWS, _COLS), lambda i: (0, 0)),
        ],
        out_specs=pl.BlockSpec((_ROWS, _COLS), lambda i: (0, 0)),
        out_shape=jax.ShapeDtypeStruct((_ROWS, _COLS), jnp.int32),
    )(probs2d, gumbel2d)


_NC = 2
_NSUB = 16
_NW = _NC * _NSUB
_CHUNK = _VPAD // _NW
_CAP = _NS
_B = 20480
_PPW = _B // _NW
_NPW = _NS // _NW


def _compact_body(tpos_hbm, vals_hbm, out_hbm, tpos_v, vals_v, sem):
    w = jax.lax.axis_index("s") * _NC + jax.lax.axis_index("c")
    rows = _CHUNK // 128
    rb = w * rows
    pltpu.sync_copy(tpos_hbm.at[pl.ds(rb, rows)], tpos_v)
    pltpu.sync_copy(vals_hbm.at[pl.ds(rb, rows)], vals_v)

    def outer(i, c):
        cps = []
        for jj in range(16):
            r = i * 16 + jj
            cps.append(pltpu.async_copy(
                vals_v.at[r],
                out_hbm.at[plsc.Indices(tpos_v.at[r], ignored_value=_NS)],
                sem))
        for cp in cps:
            cp.wait()
        return c

    jax.lax.fori_loop(0, rows // 16, outer, jnp.int32(0))


def _compact(tpos2d, vals2d):
    mesh = plsc.VectorSubcoreMesh(core_axis_name="c", subcore_axis_name="s")
    f = pl.kernel(
        _compact_body,
        out_type=jax.ShapeDtypeStruct((_NS + 8,), jnp.int32),
        mesh=mesh,
        compiler_params=pltpu.CompilerParams(needs_layout_passes=False),
        scratch_types=[
            pltpu.VMEM((_CHUNK // 128, 128), jnp.int32),
            pltpu.VMEM((_CHUNK // 128, 128), jnp.int32),
            pltpu.SemaphoreType.DMA,
        ],
    )
    return f(tpos2d, vals2d)


def _pairgather_body(emb2_hbm, yfp_hbm, sidp_hbm, gpos_hbm, gneg_hbm,
                     yidx_v, sidx_v, prow_v, nrow_v, sem):
    w = jax.lax.axis_index("s") * _NC + jax.lax.axis_index("c")
    pb = w * _PPW
    nb = w * _NPW
    for c in range(_PPW // 128):
        pltpu.sync_copy(yfp_hbm.at[pl.ds(pb + c * 128, 128)], yidx_v.at[c])
    pltpu.sync_copy(sidp_hbm.at[pl.ds(nb, _NPW)],
                    sidx_v.at[0, pl.ds(0, _NPW)])
    pltpu.sync_copy(sidp_hbm.at[pl.ds(nb, _NPW)],
                    sidx_v.at[0, pl.ds(_NPW, _NPW)])
    cps = []
    for c in range(_PPW // 128):
        cps.append(pltpu.async_copy(
            emb2_hbm.at[yidx_v.at[c]], prow_v.at[pl.ds(c * 128, 128)], sem))
    cps.append(pltpu.async_copy(emb2_hbm.at[sidx_v.at[0]], nrow_v, sem))
    for cp in cps:
        cp.wait()
    pltpu.sync_copy(prow_v, gpos_hbm.at[pl.ds(pb, _PPW)])
    pltpu.sync_copy(nrow_v.at[pl.ds(0, _NPW)], gneg_hbm.at[pl.ds(nb, _NPW)])


def _pair_gathers(emb2, yfp, sidp):
    mesh = plsc.VectorSubcoreMesh(core_axis_name="c", subcore_axis_name="s")
    f = pl.kernel(
        _pairgather_body,
        out_type=[
            jax.ShapeDtypeStruct((_B, 2 * _D), jnp.float32),
            jax.ShapeDtypeStruct((_NS, 2 * _D), jnp.float32),
        ],
        mesh=mesh,
        compiler_params=pltpu.CompilerParams(needs_layout_passes=False),
        scratch_types=[
            pltpu.VMEM((_PPW // 128, 128), jnp.int32),
            pltpu.VMEM((1, 128), jnp.int32),
            pltpu.VMEM((_PPW, 2 * _D), jnp.float32),
            pltpu.VMEM((128, 2 * _D), jnp.float32),
            pltpu.SemaphoreType.DMA,
        ],
    )
    return f(emb2, yfp, sidp)


def _loss_body(h_ref, gpos_ref, yf_ref, tp_ref, gneg_ref, sid_ref, sidc_ref,
               sp_ref, loss_ref, acc_ref, cnt_ref):
    step = pl.program_id(0)

    @pl.when(step == 0)
    def _():
        acc_ref[0, 0] = 0.0
        cnt_ref[0, 0] = 0.0

    h = h_ref[...]
    gneg = gneg_ref[...]
    parn = (sidc_ref[...] & 1) == 0
    eneg = jnp.where(parn, gneg[:, :_D], gneg[:, _D:])
    neg = jax.lax.dot_general(
        h, eneg, (((1,), (1,)), ((), ())), preferred_element_type=jnp.float32
    )
    yf = yf_ref[...]
    sid = sid_ref[...]
    logsp = jnp.log(sp_ref[...] + 1e-10)
    negl = jnp.where(yf == sid, -1e9, neg) - logsp
    zeros = jnp.zeros((_BR, _D), jnp.float32)
    hl = jnp.concatenate([h, zeros], axis=1)
    hr = jnp.concatenate([zeros, h], axis=1)
    h128 = jnp.where((yf & 1) == 0, hl, hr)
    posl = (jnp.sum(h128 * gpos_ref[...], axis=1, keepdims=True)
            - jnp.log(tp_ref[...] + 1e-10))
    m = jnp.maximum(jnp.max(negl, axis=1, keepdims=True), posl)
    s = jnp.sum(jnp.exp(negl - m), axis=1, keepdims=True) + jnp.exp(posl - m)
    per_row = m + jnp.log(s) - posl
    valid = yf != 0
    acc_ref[0, 0] += jnp.sum(jnp.where(valid, per_row, 0.0))
    cnt_ref[0, 0] += jnp.sum(valid.astype(jnp.float32))

    @pl.when(step == pl.num_programs(0) - 1)
    def _():
        loss_ref[...] = jnp.full((1, 1), acc_ref[0, 0] / cnt_ref[0, 0],
                                 dtype=jnp.float32)


def _fused_loss(h, gpos, yf, tp, gneg, sid, sidc, sp):
    n = h.shape[0]
    grid = n // _BR
    return pl.pallas_call(
        _loss_body,
        grid=(grid,),
        in_specs=[
            pl.BlockSpec((_BR, _D), lambda i: (i, 0)),
            pl.BlockSpec((_BR, 2 * _D), lambda i: (i, 0)),
            pl.BlockSpec((_BR, 1), lambda i: (i, 0)),
            pl.BlockSpec((_BR, 1), lambda i: (i, 0)),
            pl.BlockSpec((_NS, 2 * _D), lambda i: (0, 0)),
            pl.BlockSpec((1, _NS), lambda i: (0, 0)),
            pl.BlockSpec((_NS, 1), lambda i: (0, 0)),
            pl.BlockSpec((1, _NS), lambda i: (0, 0)),
        ],
        out_specs=pl.BlockSpec((1, 1), lambda i: (0, 0)),
        out_shape=jax.ShapeDtypeStruct((1, 1), jnp.float32),
        scratch_shapes=[
            pltpu.SMEM((1, 1), jnp.float32),
            pltpu.SMEM((1, 1), jnp.float32),
        ],
    )(h, gpos, yf, tp, gneg, sid, sidc, sp)


def kernel(hidden, y, emb_table, sampling_probs):
    h = hidden.reshape(-1, _D)
    yf = y.reshape(-1).astype(jnp.int32)
    probs_pad = jnp.zeros((_VPAD,), jnp.float32).at[:_VOCAB].set(sampling_probs)
    tpos = _select(
        probs_pad.reshape(_ROWS, _COLS),
        _gumbel_padded().reshape(_ROWS, _COLS),
    )
    vals = jnp.arange(_VPAD, dtype=jnp.int32)
    sid_full = _compact(tpos.reshape(_VPAD // 128, 128),
                        vals.reshape(_VPAD // 128, 128))
    sid = sid_full[:_NS]
    emb2 = emb_table.reshape(_VOCAB // 2, 2 * _D)
    gpos, gneg = _pair_gathers(emb2, yf >> 1, sid >> 1)
    tp = sampling_probs[yf]
    sp = sampling_probs[sid]
    loss = _fused_loss(
        h, gpos, yf.reshape(-1, 1), tp.reshape(-1, 1),
        gneg, sid.reshape(1, -1), sid.reshape(-1, 1), sp.reshape(1, -1),
    )
    return loss[0, 0]


# speedup vs baseline: 1.8525x; 1.8525x over previous
import jax
import jax.numpy as jnp
from jax.experimental import pallas as pl
from jax.experimental.pallas import tpu as pltpu
from jax.experimental.pallas import tpu_sc as plsc

_VOCAB = 1000000
_D = 64
_NS = 2048
_BR = 1024

_VPAD = 1024 * 1024


def _gumbel_padded():
    u = jax.random.uniform(
        jax.random.key(42), (_VOCAB,), minval=1e-10, maxval=1.0
    )
    g = -jnp.log(-jnp.log(u))
    return jnp.full((_VPAD,), -1e30, jnp.float32).at[:_VOCAB].set(g)


_ROWS = 1024
_COLS = 1024
_NCHUNK = 32


def _select_body(probs_ref, gumbel_ref, tpos_ref):
    z = jnp.log(probs_ref[...] + 1e-10) + gumbel_ref[...]
    b = jax.lax.bitcast_convert_type(z, jnp.uint32)
    key = jnp.where((b >> 31) == 1, ~b, b | jnp.uint32(0x80000000))

    row = jax.lax.broadcasted_iota(jnp.int32, (_ROWS, _COLS), 0)
    col = jax.lax.broadcasted_iota(jnp.int32, (_ROWS, _COLS), 1)
    idx = row * _COLS + col

    def bit_body(i, t):
        cand = t | jax.lax.shift_left(
            jnp.uint32(1), (31 - i).astype(jnp.uint32)
        )
        cnt = jnp.sum((key >= cand).astype(jnp.int32))
        return jnp.where(cnt >= _NS, cand, t)

    tval = jax.lax.fori_loop(0, 32, bit_body, jnp.uint32(0))

    need = _NS - jnp.sum((key > tval).astype(jnp.int32))

    def tie_body(j, iv):
        cand = iv | jax.lax.shift_left(jnp.int32(1), 19 - j)
        cnt = jnp.sum(((key == tval) & (idx <= cand)).astype(jnp.int32))
        return jnp.where(cnt <= need, cand, iv)

    ival = jax.lax.fori_loop(0, 20, tie_body, jnp.int32(0))

    sel = (key > tval) | ((key == tval) & (idx <= ival))
    self_f = sel.astype(jnp.float32)
    rowsum = jnp.sum(self_f, axis=1, keepdims=True)
    utri = (
        jax.lax.broadcasted_iota(jnp.int32, (_COLS, _COLS), 0)
        <= jax.lax.broadcasted_iota(jnp.int32, (_COLS, _COLS), 1)
    ).astype(jnp.float32)
    rowcum = jnp.dot(self_f, utri, preferred_element_type=jnp.float32)
    ltri = (
        jax.lax.broadcasted_iota(jnp.int32, (_ROWS, _ROWS), 0)
        > jax.lax.broadcasted_iota(jnp.int32, (_ROWS, _ROWS), 1)
    ).astype(jnp.float32)
    rowoff = jnp.dot(ltri, rowsum, preferred_element_type=jnp.float32)
    grank = (rowoff + rowcum - 1.0).astype(jnp.int32)
    tpos_ref[...] = jnp.where(sel, grank, _NS)


def _select(probs2d, gumbel2d):
    return pl.pallas_call(
        _select_body,
        grid=(1,),
        in_specs=[
            pl.BlockSpec((_ROWS, _COLS), lambda i: (0, 0)),
            pl.BlockSpec((_ROWS, _COLS), lambda i: (0, 0)),
        ],
        out_specs=pl.BlockSpec((_ROWS, _COLS), lambda i: (0, 0)),
        out_shape=jax.ShapeDtypeStruct((_ROWS, _COLS), jnp.int32),
    )(probs2d, gumbel2d)


_NC = 2
_NSUB = 16
_NW = _NC * _NSUB
_CHUNK = _VPAD // _NW
_CAP = _NS
_B = 20480
_PPW = _B // _NW
_NPW = _NS // _NW


def _compact_body(tpos_hbm, vals_hbm, out_hbm, tpos_v, vals_v, sem):
    w = jax.lax.axis_index("s") * _NC + jax.lax.axis_index("c")
    rows = _CHUNK // 128
    rb = w * rows
    pltpu.sync_copy(tpos_hbm.at[pl.ds(rb, rows)], tpos_v)
    pltpu.sync_copy(vals_hbm.at[pl.ds(rb, rows)], vals_v)

    def outer(i, c):
        cps = []
        for jj in range(16):
            r = i * 16 + jj
            cps.append(pltpu.async_copy(
                vals_v.at[r],
                out_hbm.at[plsc.Indices(tpos_v.at[r], ignored_value=_NS)],
                sem))
        for cp in cps:
            cp.wait()
        return c

    jax.lax.fori_loop(0, rows // 16, outer, jnp.int32(0))


def _compact(tpos2d, vals2d):
    mesh = plsc.VectorSubcoreMesh(core_axis_name="c", subcore_axis_name="s")
    f = pl.kernel(
        _compact_body,
        out_type=jax.ShapeDtypeStruct((_NS + 8,), jnp.int32),
        mesh=mesh,
        compiler_params=pltpu.CompilerParams(needs_layout_passes=False),
        scratch_types=[
            pltpu.VMEM((_CHUNK // 128, 128), jnp.int32),
            pltpu.VMEM((_CHUNK // 128, 128), jnp.int32),
            pltpu.SemaphoreType.DMA,
        ],
    )
    return f(tpos2d, vals2d)


def _loss_body(h_ref, epos_ref, yf_ref, tp_ref, eneg_ref, sid_ref, sp_ref,
               loss_ref, acc_ref, cnt_ref):
    step = pl.program_id(0)

    @pl.when(step == 0)
    def _():
        acc_ref[0, 0] = 0.0
        cnt_ref[0, 0] = 0.0

    h = h_ref[...]
    eneg = eneg_ref[...]
    neg = jax.lax.dot_general(
        h, eneg, (((1,), (1,)), ((), ())), preferred_element_type=jnp.float32
    )
    yf = yf_ref[...]
    sid = sid_ref[...]
    logsp = jnp.log(sp_ref[...] + 1e-10)
    negl = jnp.where(yf == sid, -1e9, neg) - logsp
    posl = (jnp.sum(h * epos_ref[...], axis=1, keepdims=True)
            - jnp.log(tp_ref[...] + 1e-10))
    m = jnp.maximum(jnp.max(negl, axis=1, keepdims=True), posl)
    s = jnp.sum(jnp.exp(negl - m), axis=1, keepdims=True) + jnp.exp(posl - m)
    per_row = m + jnp.log(s) - posl
    valid = yf != 0
    acc_ref[0, 0] += jnp.sum(jnp.where(valid, per_row, 0.0))
    cnt_ref[0, 0] += jnp.sum(valid.astype(jnp.float32))

    @pl.when(step == pl.num_programs(0) - 1)
    def _():
        loss_ref[...] = jnp.full((1, 1), acc_ref[0, 0] / cnt_ref[0, 0],
                                 dtype=jnp.float32)


def _fused_loss(h, epos, yf, tp, eneg, sid, sp):
    n = h.shape[0]
    grid = n // _BR
    return pl.pallas_call(
        _loss_body,
        grid=(grid,),
        in_specs=[
            pl.BlockSpec((_BR, _D), lambda i: (i, 0)),
            pl.BlockSpec((_BR, _D), lambda i: (i, 0)),
            pl.BlockSpec((_BR, 1), lambda i: (i, 0)),
            pl.BlockSpec((_BR, 1), lambda i: (i, 0)),
            pl.BlockSpec((_NS, _D), lambda i: (0, 0)),
            pl.BlockSpec((1, _NS), lambda i: (0, 0)),
            pl.BlockSpec((1, _NS), lambda i: (0, 0)),
        ],
        out_specs=pl.BlockSpec((1, 1), lambda i: (0, 0)),
        out_shape=jax.ShapeDtypeStruct((1, 1), jnp.float32),
        scratch_shapes=[
            pltpu.SMEM((1, 1), jnp.float32),
            pltpu.SMEM((1, 1), jnp.float32),
        ],
    )(h, epos, yf, tp, eneg, sid, sp)


def kernel(hidden, y, emb_table, sampling_probs):
    h = hidden.reshape(-1, _D)
    yf = y.reshape(-1).astype(jnp.int32)
    probs_pad = jnp.zeros((_VPAD,), jnp.float32).at[:_VOCAB].set(sampling_probs)
    tpos = _select(
        probs_pad.reshape(_ROWS, _COLS),
        _gumbel_padded().reshape(_ROWS, _COLS),
    )
    vals = jnp.arange(_VPAD, dtype=jnp.int32)
    sid_full = _compact(tpos.reshape(_VPAD // 128, 128),
                        vals.reshape(_VPAD // 128, 128))
    sid = sid_full[:_NS]
    epos = emb_table[yf]
    eneg = emb_table[sid]
    tp = sampling_probs[yf]
    sp = sampling_probs[sid]
    loss = _fused_loss(
        h, epos, yf.reshape(-1, 1), tp.reshape(-1, 1),
        eneg, sid.reshape(1, -1), sp.reshape(1, -1),
    )
    return loss[0, 0]


# speedup vs baseline: 1.8748x; 1.0120x over previous
import jax
import jax.numpy as jnp
from jax.experimental import pallas as pl
from jax.experimental.pallas import tpu as pltpu
from jax.experimental.pallas import tpu_sc as plsc

_VOCAB = 1000000
_D = 64
_NS = 2048
_BR = 1280

_VPAD = 1024 * 1024


def _gumbel_padded():
    u = jax.random.uniform(
        jax.random.key(42), (_VOCAB,), minval=1e-10, maxval=1.0
    )
    g = -jnp.log(-jnp.log(u))
    return jnp.full((_VPAD,), -1e30, jnp.float32).at[:_VOCAB].set(g)


_ROWS = 1024
_COLS = 1024
_NCHUNK = 32


def _select_body(probs_ref, gumbel_ref, tpos_ref):
    z = jnp.log(probs_ref[...] + 1e-10) + gumbel_ref[...]
    b = jax.lax.bitcast_convert_type(z, jnp.uint32)
    key = jnp.where((b >> 31) == 1, ~b, b | jnp.uint32(0x80000000))

    row = jax.lax.broadcasted_iota(jnp.int32, (_ROWS, _COLS), 0)
    col = jax.lax.broadcasted_iota(jnp.int32, (_ROWS, _COLS), 1)
    idx = row * _COLS + col

    def bit_body(i, t):
        cand = t | jax.lax.shift_left(
            jnp.uint32(1), (31 - i).astype(jnp.uint32)
        )
        cnt = jnp.sum((key >= cand).astype(jnp.int32))
        return jnp.where(cnt >= _NS, cand, t)

    tval = jax.lax.fori_loop(0, 32, bit_body, jnp.uint32(0))

    need = _NS - jnp.sum((key > tval).astype(jnp.int32))

    def tie_body(j, iv):
        cand = iv | jax.lax.shift_left(jnp.int32(1), 19 - j)
        cnt = jnp.sum(((key == tval) & (idx <= cand)).astype(jnp.int32))
        return jnp.where(cnt <= need, cand, iv)

    ival = jax.lax.fori_loop(0, 20, tie_body, jnp.int32(0))

    sel = (key > tval) | ((key == tval) & (idx <= ival))
    self_f = sel.astype(jnp.float32)
    rowsum = jnp.sum(self_f, axis=1, keepdims=True)
    utri = (
        jax.lax.broadcasted_iota(jnp.int32, (_COLS, _COLS), 0)
        <= jax.lax.broadcasted_iota(jnp.int32, (_COLS, _COLS), 1)
    ).astype(jnp.float32)
    rowcum = jnp.dot(self_f, utri, preferred_element_type=jnp.float32)
    ltri = (
        jax.lax.broadcasted_iota(jnp.int32, (_ROWS, _ROWS), 0)
        > jax.lax.broadcasted_iota(jnp.int32, (_ROWS, _ROWS), 1)
    ).astype(jnp.float32)
    rowoff = jnp.dot(ltri, rowsum, preferred_element_type=jnp.float32)
    grank = (rowoff + rowcum - 1.0).astype(jnp.int32)
    tpos_ref[...] = jnp.where(sel, grank, _NS)


def _select(probs2d, gumbel2d):
    return pl.pallas_call(
        _select_body,
        grid=(1,),
        in_specs=[
            pl.BlockSpec((_ROWS, _COLS), lambda i: (0, 0)),
            pl.BlockSpec((_ROWS, _COLS), lambda i: (0, 0)),
        ],
        out_specs=pl.BlockSpec((_ROWS, _COLS), lambda i: (0, 0)),
        out_shape=jax.ShapeDtypeStruct((_ROWS, _COLS), jnp.int32),
    )(probs2d, gumbel2d)


_NC = 2
_NSUB = 16
_NW = _NC * _NSUB
_CHUNK = _VPAD // _NW
_CAP = _NS
_B = 20480
_PPW = _B // _NW
_NPW = _NS // _NW


def _compact_body(tpos_hbm, vals_hbm, out_hbm, tpos_v, vals_v, sem):
    w = jax.lax.axis_index("s") * _NC + jax.lax.axis_index("c")
    rows = _CHUNK // 128
    rb = w * rows
    pltpu.sync_copy(tpos_hbm.at[pl.ds(rb, rows)], tpos_v)
    pltpu.sync_copy(vals_hbm.at[pl.ds(rb, rows)], vals_v)

    def outer(i, c):
        cps = []
        for jj in range(16):
            r = i * 16 + jj
            cps.append(pltpu.async_copy(
                vals_v.at[r],
                out_hbm.at[plsc.Indices(tpos_v.at[r], ignored_value=_NS)],
                sem))
        for cp in cps:
            cp.wait()
        return c

    jax.lax.fori_loop(0, rows // 16, outer, jnp.int32(0))


def _compact(tpos2d, vals2d):
    mesh = plsc.VectorSubcoreMesh(core_axis_name="c", subcore_axis_name="s")
    f = pl.kernel(
        _compact_body,
        out_type=jax.ShapeDtypeStruct((_NS + 8,), jnp.int32),
        mesh=mesh,
        compiler_params=pltpu.CompilerParams(needs_layout_passes=False),
        scratch_types=[
            pltpu.VMEM((_CHUNK // 128, 128), jnp.int32),
            pltpu.VMEM((_CHUNK // 128, 128), jnp.int32),
            pltpu.SemaphoreType.DMA,
        ],
    )
    return f(tpos2d, vals2d)


def _loss_body(h_ref, epos_ref, yf_ref, tp_ref, eneg_ref, sid_ref, sp_ref,
               loss_ref, acc_ref, cnt_ref):
    step = pl.program_id(0)

    @pl.when(step == 0)
    def _():
        acc_ref[0, 0] = 0.0
        cnt_ref[0, 0] = 0.0

    h = h_ref[...].reshape(_BR, _D)
    eneg = eneg_ref[...]
    neg = jax.lax.dot_general(
        h, eneg, (((1,), (1,)), ((), ())), preferred_element_type=jnp.float32
    )
    yf = yf_ref[...]
    sid = sid_ref[...]
    logsp = jnp.log(sp_ref[...] + 1e-10)
    negl = jnp.where(yf == sid, -1e9, neg) - logsp
    posl = (jnp.sum(h * epos_ref[...], axis=1, keepdims=True)
            - jnp.log(tp_ref[...] + 1e-10))
    m = jnp.maximum(jnp.max(negl, axis=1, keepdims=True), posl)
    s = jnp.sum(jnp.exp(negl - m), axis=1, keepdims=True) + jnp.exp(posl - m)
    per_row = m + jnp.log(s) - posl
    valid = yf != 0
    acc_ref[0, 0] += jnp.sum(jnp.where(valid, per_row, 0.0))
    cnt_ref[0, 0] += jnp.sum(valid.astype(jnp.float32))

    @pl.when(step == pl.num_programs(0) - 1)
    def _():
        loss_ref[...] = jnp.full((1, 1), acc_ref[0, 0] / cnt_ref[0, 0],
                                 dtype=jnp.float32)


def _fused_loss(hidden, epos, yf, tp, eneg, sid, sp):
    n = epos.shape[0]
    grid = n // _BR
    return pl.pallas_call(
        _loss_body,
        grid=(grid,),
        in_specs=[
            pl.BlockSpec((_BR // 20, 20, _D), lambda i: (i, 0, 0)),
            pl.BlockSpec((_BR, _D), lambda i: (i, 0)),
            pl.BlockSpec((_BR, 1), lambda i: (i, 0)),
            pl.BlockSpec((_BR, 1), lambda i: (i, 0)),
            pl.BlockSpec((_NS, _D), lambda i: (0, 0)),
            pl.BlockSpec((1, _NS), lambda i: (0, 0)),
            pl.BlockSpec((1, _NS), lambda i: (0, 0)),
        ],
        out_specs=pl.BlockSpec((1, 1), lambda i: (0, 0)),
        out_shape=jax.ShapeDtypeStruct((1, 1), jnp.float32),
        scratch_shapes=[
            pltpu.SMEM((1, 1), jnp.float32),
            pltpu.SMEM((1, 1), jnp.float32),
        ],
    )(hidden, epos, yf, tp, eneg, sid, sp)


def kernel(hidden, y, emb_table, sampling_probs):
    yf = y.reshape(-1).astype(jnp.int32)
    probs_pad = jnp.zeros((_VPAD,), jnp.float32).at[:_VOCAB].set(sampling_probs)
    tpos = _select(
        probs_pad.reshape(_ROWS, _COLS),
        _gumbel_padded().reshape(_ROWS, _COLS),
    )
    vals = jnp.arange(_VPAD, dtype=jnp.int32)
    sid_full = _compact(tpos.reshape(_VPAD // 128, 128),
                        vals.reshape(_VPAD // 128, 128))
    sid = sid_full[:_NS]
    epos = emb_table[yf]
    eneg = emb_table[sid]
    tp = sampling_probs[yf]
    sp = sampling_probs[sid]
    loss = _fused_loss(
        hidden, epos, yf.reshape(-1, 1), tp.reshape(-1, 1),
        eneg, sid.reshape(1, -1), sp.reshape(1, -1),
    )
    return loss[0, 0]
